# baseline (speedup 1.0000x reference)
import functools

import jax
import jax.numpy as jnp
from jax import lax
from jax.experimental import pallas as pl
from jax.experimental.pallas import tpu as pltpu
from jax.experimental.pallas import tpu_sc as plsc

VOCAB = 1000000
DIM = 32
B = 16384
NEG = 20

NC = 2
NS = 16
L = 16
NW = NC * NS
BPW = B // NW
CR = 80
NCH = (BPW * NEG) // CR
NBUF = 2

_mesh = plsc.VectorSubcoreMesh(core_axis_name="c", subcore_axis_name="s")


@functools.partial(
    pl.kernel,
    out_type=[
        jax.ShapeDtypeStruct((B,), jnp.float32),
        jax.ShapeDtypeStruct((B, NEG), jnp.float32),
    ],
    mesh=_mesh,
    compiler_params=pltpu.CompilerParams(
        needs_layout_passes=False, use_tc_tiling_on_sc=False),
    scratch_types=[
        pltpu.VMEM((4, 128), jnp.int32),
        pltpu.VMEM((4, 128), jnp.int32),
        pltpu.VMEM((NCH, CR), jnp.int32),
        pltpu.VMEM((BPW, DIM), jnp.float32),
        pltpu.VMEM((BPW, DIM), jnp.float32),
        pltpu.VMEM((NBUF, CR, DIM), jnp.float32),
        pltpu.VMEM((BPW,), jnp.float32),
        pltpu.VMEM((BPW, NEG), jnp.float32),
        pltpu.SemaphoreType.DMA,
        pltpu.SemaphoreType.DMA,
    ],
)
def _sc_scores(cw_hbm, xw_hbm, nw_hbm, ctab_hbm, xtab_hbm,
               pos_out, neg_out,
               idx_c, idx_x, idx_n, c_rows, x_rows, n_ring,
               pos_buf, neg_buf, sem_cx, sem_n):
    wid = lax.axis_index("s") * NC + lax.axis_index("c")
    base = wid * BPW
    iota = lax.iota(jnp.int32, L)

    pltpu.sync_copy(cw_hbm.at[pl.ds(wid * 4, 4)], idx_c)
    pltpu.sync_copy(xw_hbm.at[pl.ds(wid * 4, 4)], idx_x)
    pltpu.sync_copy(nw_hbm.at[pl.ds(wid * NCH, NCH)], idx_n)

    cps = []
    for j in range(4):
        cps.append(pltpu.async_copy(
            ctab_hbm.at[idx_c.at[j]], c_rows.at[pl.ds(j * 128, 128)], sem_cx))
        cps.append(pltpu.async_copy(
            xtab_hbm.at[idx_x.at[j]], x_rows.at[pl.ds(j * 128, 128)], sem_cx))
    for u in range(NBUF):
        pltpu.async_copy(xtab_hbm.at[idx_n.at[u]], n_ring.at[u], sem_n)
    for cp in cps:
        cp.wait()

    @pl.loop(0, BPW // L)
    def _pos(grp):
        bl = grp * L + iota
        acc = jnp.zeros((L,), jnp.float32)
        for d in range(DIM):
            dd = jnp.full((L,), d, jnp.int32)
            cv = plsc.load_gather(c_rows, [bl, dd])
            xv = plsc.load_gather(x_rows, [bl, dd])
            acc = acc + cv * xv
        pos_buf[pl.ds(grp * L, L)] = acc

    @pl.loop(0, NCH, step=NBUF)
    def _neg(g0):
        for u in range(NBUF):
            g = g0 + u
            pltpu.make_async_copy(
                xtab_hbm.at[idx_n.at[g]], n_ring.at[u], sem_n).wait()
            for grp in range(CR // L):
                rloc = grp * L + iota
                r = g * CR + rloc
                bl = r // NEG
                kl = r - bl * NEG
                acc = jnp.zeros((L,), jnp.float32)
                for d in range(DIM):
                    dd = jnp.full((L,), d, jnp.int32)
                    nv = plsc.load_gather(n_ring.at[u], [rloc, dd])
                    cv = plsc.load_gather(c_rows, [bl, dd])
                    acc = acc + nv * cv
                plsc.store_scatter(neg_buf, [bl, kl], acc)

            @pl.when(g + NBUF < NCH)
            def _refill():
                pltpu.async_copy(
                    xtab_hbm.at[idx_n.at[g + NBUF]], n_ring.at[u], sem_n)

    pltpu.sync_copy(pos_buf, pos_out.at[pl.ds(base, BPW)])
    pltpu.sync_copy(neg_buf, neg_out.at[pl.ds(base, BPW)])


def _loss_body(pos_ref, neg_ref, out_ref):
    pos = pos_ref[...]
    neg = neg_ref[...]
    pos_loss = jnp.log(1.0 / (1.0 + jnp.exp(-pos)) + 1e-10)
    neg_loss = jnp.log(1.0 / (1.0 + jnp.exp(neg)) + 1e-10)
    out_ref[0, 0] = -(jnp.sum(pos_loss) + jnp.sum(neg_loss)) / B


_loss = pl.pallas_call(
    _loss_body,
    out_shape=jax.ShapeDtypeStruct((1, 1), jnp.float32),
    out_specs=pl.BlockSpec(memory_space=pltpu.SMEM),
)


def kernel(center_words, context_words, negative_words, center_table, context_table):
    cw = center_words.astype(jnp.int32).reshape(B // 128, 128)
    xw = context_words.astype(jnp.int32).reshape(B // 128, 128)
    nw = negative_words.astype(jnp.int32).reshape((B * NEG) // CR, CR)
    pos, neg = _sc_scores(cw, xw, nw, center_table, context_table)
    return _loss(pos.reshape(128, 128), neg.reshape((B * NEG) // 128, 128))[0, 0]
